# baseline (speedup 1.0000x reference)
import jax
import jax.numpy as jnp
from jax import lax
from jax.experimental import pallas as pl
from jax.experimental.pallas import tpu as pltpu
from jax.experimental.pallas import tpu_sc as plsc

N_USERS = 5000
N_ITEMS = 5000
N = N_USERS + N_ITEMS
EMB = 128
HALF = EMB // 2
E = 320000
LAYERS = 3

NS = 16
EPT = E // NS
CH = 128
NB = 3
NCH = 159
EPA = NCH * CH
NP = 10240
RPT = NP // NS
ZR = 128


def _body(x0s, src_hbm, dst_hbm, w_hbm, y1, y2, y3,
          srcf, dstf, wf, rows0, rows1, rows2, acc, gsem):
    rows = (rows0, rows1, rows2)
    c = lax.axis_index("c")
    s = lax.axis_index("s")
    base = s * EPT
    row0 = s * RPT

    zi = jnp.zeros((16,), jnp.int32)
    zf = jnp.zeros((16,), jnp.float32)

    pltpu.sync_copy(src_hbm.at[pl.ds(base, EPT)], srcf.at[pl.ds(0, EPT)])
    pltpu.sync_copy(dst_hbm.at[pl.ds(base, EPT)], dstf.at[pl.ds(0, EPT)])
    pltpu.sync_copy(w_hbm.at[pl.ds(base, EPT)], wf.at[pl.ds(0, EPT)])
    for t in range((EPA - EPT) // 16):
        sl = pl.ds(EPT + t * 16, 16)
        srcf[sl] = zi
        dstf[sl] = zi
        wf[sl] = zf

    srcs = (x0s, y1, y2)
    outs = (y1, y2, y3)
    for L in range(LAYERS):
        xsrc = srcs[L].at[c]
        def zrow(r, carry):
            for k in range(HALF // 16):
                rows0[r, pl.ds(k * 16, 16)] = zf
            return carry
        lax.fori_loop(0, ZR, zrow, 0)
        for k in range(RPT // ZR):
            pltpu.sync_copy(rows0.at[pl.ds(0, ZR)],
                            acc.at[pl.ds(row0 + k * ZR, ZR)])
        plsc.subcore_barrier()

        def scale(buf, e0):
            def scale_group(g, carry2):
                wv16 = wf[pl.ds(e0 + g * 16, 16)]
                for r16 in range(16):
                    wv = jnp.full((16,), wv16[r16], jnp.float32)
                    r = g * 16 + r16
                    for k in range(HALF // 16):
                        sl = pl.ds(k * 16, 16)
                        buf[r, sl] = buf[r, sl] * wv
                return carry2
            lax.fori_loop(0, CH // 16, scale_group, 0)

        def group(i, carry):
            eg = i * (NB * CH)
            descs = []
            for b in range(NB):
                e0 = eg + b * CH
                descs.append(pltpu.async_copy(
                    xsrc.at[srcf.at[pl.ds(e0, CH)]], rows[b], gsem))
            for b in range(NB):
                e0 = eg + b * CH
                descs[b].wait()
                scale(rows[b], e0)
                pltpu.sync_copy(rows[b], acc.at[dstf.at[pl.ds(e0, CH)]],
                                add=True)
            return carry
        lax.fori_loop(0, NCH // NB, group, 0)
        plsc.subcore_barrier()

        pltpu.sync_copy(acc.at[pl.ds(row0, RPT)],
                        outs[L].at[c].at[pl.ds(row0, RPT)])
        plsc.subcore_barrier()


def _propagate(x0s, src, dst, w):
    mesh = plsc.VectorSubcoreMesh(core_axis_name="c", subcore_axis_name="s")
    fn = pl.kernel(
        _body,
        out_type=[jax.ShapeDtypeStruct((2, NP, HALF), jnp.float32)] * LAYERS,
        mesh=mesh,
        scratch_types=[
            pltpu.VMEM((EPA,), jnp.int32),
            pltpu.VMEM((EPA,), jnp.int32),
            pltpu.VMEM((EPA,), jnp.float32),
            pltpu.VMEM((CH, HALF), jnp.float32),
            pltpu.VMEM((CH, HALF), jnp.float32),
            pltpu.VMEM((CH, HALF), jnp.float32),
            pltpu.VMEM_SHARED((NP, HALF), jnp.float32),
            pltpu.SemaphoreType.DMA,
        ],
        compiler_params=pltpu.CompilerParams(use_tc_tiling_on_sc=False),
    )
    return fn(x0s, src, dst, w)


def kernel(user_emb, item_emb, edge_index, edge_weight):
    x0 = jnp.concatenate([user_emb, item_emb], axis=0)
    x0p = jnp.pad(x0, ((0, NP - N), (0, 0)))
    x0s = jnp.stack([x0p[:, :HALF], x0p[:, HALF:]])
    ys = _propagate(x0s, edge_index[0], edge_index[1], edge_weight)
    layers = [x0] + [jnp.concatenate([y[0, :N], y[1, :N]], axis=-1)
                     for y in ys]
    return jnp.stack(layers)
